# Initial kernel scaffold; baseline (speedup 1.0000x reference)
import jax
import jax.numpy as jnp
from jax.experimental import pallas as pl


def kernel(hidden_states, router_weight, e_score_correction_bias, gate_w, up_w, down_w, shared_gate_w, shared_up_w, shared_down_w):
    raise NotImplementedError("write your pallas kernel here")


# speedup vs baseline: 1.3893x; 1.3893x over previous
import functools

import jax
import jax.numpy as jnp
from jax import lax
from jax.experimental import pallas as pl
from jax.experimental.pallas import tpu as pltpu

E = 8
TOP_K = 2
N_GROUP = 4
EPG = E // N_GROUP
TOPK_GROUP = 2
ROUTED_SCALE = 2.5
Z_COEF = 0.001
BAL_COEF = 0.001

BT = 256


def _argmax_first(s, iota, big):
    m = jnp.max(s, axis=-1, keepdims=True)
    i = jnp.min(jnp.where(s == m, iota, big), axis=-1, keepdims=True)
    return m, i


def _router_body(nt_total, x_ref, wr_ref, bias_ref, cmb_ref, aux_ref,
                 wsum_ref, cnt_ref, z_ref):
    nt = pl.program_id(0)
    xb = x_ref[...]
    logits = lax.dot_general(xb, wr_ref[...], (((1,), (1,)), ((), ())),
                             preferred_element_type=jnp.float32)
    scores = jax.nn.sigmoid(logits)
    sfc = scores + bias_ref[...]

    iota_e = lax.broadcasted_iota(jnp.int32, (BT, E), 1)
    iota_g = lax.broadcasted_iota(jnp.int32, (BT, N_GROUP), 1)

    gmap = (lax.broadcasted_iota(jnp.int32, (E, N_GROUP), 0) // EPG ==
            lax.broadcasted_iota(jnp.int32, (E, N_GROUP), 1)).astype(jnp.float32)
    gs = jnp.dot(sfc, gmap, preferred_element_type=jnp.float32,
                 precision=lax.Precision.HIGHEST)

    _, g1 = _argmax_first(gs, iota_g, N_GROUP)
    gs2 = jnp.where(iota_g == g1, -jnp.inf, gs)
    _, g2 = _argmax_first(gs2, iota_g, N_GROUP)

    eg = iota_e // EPG
    emask = (eg == g1) | (eg == g2)
    sm = jnp.where(emask, sfc, -1.0)

    _, e1 = _argmax_first(sm, iota_e, E)
    sm2 = jnp.where(iota_e == e1, -jnp.inf, sm)
    _, e2 = _argmax_first(sm2, iota_e, E)

    sel1 = (iota_e == e1)
    sel2 = (iota_e == e2)
    w1 = jnp.sum(jnp.where(sel1, scores, 0.0), axis=-1, keepdims=True)
    w2 = jnp.sum(jnp.where(sel2, scores, 0.0), axis=-1, keepdims=True)
    denom = w1 + w2 + 1e-20
    w1 = w1 / denom * ROUTED_SCALE
    w2 = w2 / denom * ROUTED_SCALE

    iota16 = lax.broadcasted_iota(jnp.int32, (BT, 16), 1)
    e1b = jnp.broadcast_to(e1, (BT, 16))
    e2b = jnp.broadcast_to(e2, (BT, 16))
    cmb = (jnp.where(iota16 == e1b, jnp.broadcast_to(w1, (BT, 16)), 0.0) +
           jnp.where(iota16 == e2b, jnp.broadcast_to(w2, (BT, 16)), 0.0) +
           jnp.where(iota16 == E, 1.0, 0.0))
    cmb_ref[...] = cmb

    @pl.when(nt == 0)
    def _():
        wsum_ref[...] = jnp.zeros_like(wsum_ref)
        cnt_ref[...] = jnp.zeros_like(cnt_ref)
        z_ref[0] = 0.0

    wsum_ref[...] += jnp.sum(cmb[:, :16], axis=0, keepdims=True)
    cnt_ref[...] += jnp.sum(
        jnp.where(iota16 == e1b, 1.0, 0.0) + jnp.where(iota16 == e2b, 1.0, 0.0),
        axis=0, keepdims=True)
    mx = jnp.max(logits, axis=-1, keepdims=True)
    lse = jnp.log(jnp.sum(jnp.exp(logits - mx), axis=-1, keepdims=True)) + mx
    z_ref[0] += jnp.sum(lse * lse)

    @pl.when(nt == nt_total - 1)
    def _():
        t_tot = jnp.float32(nt_total * BT)
        col = lax.broadcasted_iota(jnp.int32, (1, 16), 1)
        keep = col < E
        mean_load = jnp.where(keep, wsum_ref[...], 0.0) / t_tot
        freq = jnp.where(keep, cnt_ref[...], 0.0) / t_tot
        balance = E * jnp.sum(mean_load * freq)
        aux_ref[...] = jnp.broadcast_to(
            BAL_COEF * balance + Z_COEF * (z_ref[0] / t_tot), (1, 1))


def _ffn_body(d, i_dim, x_ref, gw_ref, uw_ref, dw_ref, cmb_ref, out_ref, acc_ref):
    e = pl.program_id(0)
    nt = pl.program_id(1)
    xb = x_ref[...]
    g = lax.dot_general(xb, gw_ref[0], (((1,), (1,)), ((), ())),
                        preferred_element_type=jnp.float32)
    u = lax.dot_general(xb, uw_ref[0], (((1,), (1,)), ((), ())),
                        preferred_element_type=jnp.float32)
    h = g * jax.nn.sigmoid(g) * u
    y = lax.dot_general(h, dw_ref[0], (((1,), (1,)), ((), ())),
                        preferred_element_type=jnp.float32)
    iota16 = lax.broadcasted_iota(jnp.int32, (BT, 16), 1)
    c = jnp.sum(jnp.where(iota16 == e, cmb_ref[...], 0.0), axis=-1,
                keepdims=True)
    cy = c * y
    sl = pl.ds(nt * BT, BT)

    @pl.when(e == 0)
    def _():
        acc_ref[sl, :] = cy

    @pl.when((e > 0) & (e < E))
    def _():
        acc_ref[sl, :] += cy

    @pl.when(e == E)
    def _():
        out_ref[...] = acc_ref[sl, :] + cy


def kernel(hidden_states, router_weight, e_score_correction_bias, gate_w,
           up_w, down_w, shared_gate_w, shared_up_w, shared_down_w):
    bsz, seq, d = hidden_states.shape
    t = bsz * seq
    i_dim = gate_w.shape[1]
    nt = t // BT
    x = hidden_states.reshape(t, d)
    bias2 = e_score_correction_bias.reshape(1, E)

    cmb, aux = pl.pallas_call(
        functools.partial(_router_body, nt),
        grid=(nt,),
        in_specs=[
            pl.BlockSpec((BT, d), lambda n: (n, 0)),
            pl.BlockSpec((E, d), lambda n: (0, 0)),
            pl.BlockSpec((1, E), lambda n: (0, 0)),
        ],
        out_specs=[
            pl.BlockSpec((BT, 16), lambda n: (n, 0)),
            pl.BlockSpec((1, 1), lambda n: (0, 0)),
        ],
        out_shape=[
            jax.ShapeDtypeStruct((t, 16), jnp.float32),
            jax.ShapeDtypeStruct((1, 1), jnp.float32),
        ],
        scratch_shapes=[
            pltpu.VMEM((1, 16), jnp.float32),
            pltpu.VMEM((1, 16), jnp.float32),
            pltpu.SMEM((1,), jnp.float32),
        ],
    )(x, router_weight, bias2)

    gw_all = jnp.concatenate([gate_w, shared_gate_w[None]], axis=0)
    uw_all = jnp.concatenate([up_w, shared_up_w[None]], axis=0)
    dw_all = jnp.concatenate([down_w, shared_down_w[None]], axis=0)

    out = pl.pallas_call(
        functools.partial(_ffn_body, d, i_dim),
        grid=(E + 1, nt),
        in_specs=[
            pl.BlockSpec((BT, d), lambda e, n: (n, 0)),
            pl.BlockSpec((1, i_dim, d), lambda e, n: (e, 0, 0)),
            pl.BlockSpec((1, i_dim, d), lambda e, n: (e, 0, 0)),
            pl.BlockSpec((1, d, i_dim), lambda e, n: (e, 0, 0)),
            pl.BlockSpec((BT, 16), lambda e, n: (n, 0)),
        ],
        out_specs=pl.BlockSpec((BT, d), lambda e, n: (n, 0)),
        out_shape=jax.ShapeDtypeStruct((t, d), jnp.float32),
        scratch_shapes=[pltpu.VMEM((t, d), jnp.float32)],
    )(x, gw_all, uw_all, dw_all, cmb)

    return out.reshape(bsz, seq, d), aux[0, 0]
